# baseline (speedup 1.0000x reference)
import functools

import jax
import jax.numpy as jnp
from jax import lax
from jax.experimental import pallas as pl
from jax.experimental.pallas import tpu as pltpu
from jax.experimental.pallas import tpu_sc as plsc

B = 128
C = 64
T = 4096
NB = 8
CC = C * C
NW = 32
BPW = B // NW
L = 16


def _idx_body(pred_ref, act_ref, pidx_ref, aidx_ref):
    for i in range(NB):
        for r, o in ((pred_ref, pidx_ref), (act_ref, aidx_ref)):
            x = r[i]
            m = jnp.max(x, axis=0, keepdims=True)
            ci = lax.broadcasted_iota(jnp.int32, x.shape, 0)
            o[i, :, :] = jnp.min(jnp.where(x == m, ci, C), axis=0,
                                 keepdims=True)


def _sc_hist_make():
    mesh = plsc.VectorSubcoreMesh(core_axis_name="c", subcore_axis_name="s")

    @functools.partial(
        pl.kernel, mesh=mesh,
        compiler_params=pltpu.CompilerParams(needs_layout_passes=False),
        out_type=jax.ShapeDtypeStruct((B, 2, CC), jnp.float32),
        scratch_types=[
            pltpu.VMEM((T,), jnp.int32),
            pltpu.VMEM((CC,), jnp.float32),
        ],
    )
    def _sc_hist(pidx_hbm, aidx_hbm, out_hbm, idx_v, bins_v):
        wid = lax.axis_index("s") * 2 + lax.axis_index("c")
        zeros16 = jnp.zeros((L,), jnp.float32)
        ones16 = jnp.ones((L,), jnp.float32)
        lane_iota = lax.iota(jnp.int32, L)
        for bb in range(BPW):
            b = wid * BPW + bb
            for src_i in range(2):
                hbm = pidx_hbm if src_i == 0 else aidx_hbm
                pltpu.sync_copy(hbm.at[b, 0], idx_v)

                def zstep(t, _):
                    bins_v[pl.ds(t * L, L)] = zeros16
                    return 0
                lax.fori_loop(0, CC // L, zstep, 0)

                def step(t, _):
                    base = t * L
                    cur = idx_v[pl.ds(base, L)]
                    nxt = idx_v[pl.ds(base + 1, L)]
                    code = cur * C + nxt
                    val = jnp.where(cur != nxt, 1.0, 0.0)
                    plsc.addupdate_scatter(bins_v, [code], val)
                    return 0
                lax.fori_loop(0, T // L - 1, step, 0)
                cur = idx_v[pl.ds(T - L - 1, L)]
                nxt = idx_v[pl.ds(T - L, L)]
                code = cur * C + nxt
                val = jnp.where((cur != nxt) & (lane_iota > 0), 1.0, 0.0)
                plsc.addupdate_scatter(bins_v, [code], val)
                pltpu.sync_copy(bins_v, out_hbm.at[b, src_i])

    return _sc_hist


def _loss_body(cnt_ref, bce_ref, sq_ref, cnt_out_ref):
    ri = lax.broadcasted_iota(jnp.int32, (C, C), 0)
    cj = lax.broadcasted_iota(jnp.int32, (C, C), 1)
    offdiag = (ri != cj).astype(jnp.float32)

    p_counts = cnt_ref[0, 0] * offdiag
    t_counts = cnt_ref[0, 1] * offdiag

    true_adj = t_counts / (jnp.sum(t_counts, axis=1, keepdims=True) + 1e-8)
    pred_adj = p_counts / (jnp.sum(p_counts, axis=1, keepdims=True) + 1e-8)

    gt = (t_counts > 0).astype(jnp.float32)
    dense_pred = jnp.tanh(pred_adj)
    log_p = jnp.maximum(jnp.log(dense_pred), -100.0)
    log_1mp = jnp.maximum(jnp.log1p(-dense_pred), -100.0)
    bce = -jnp.sum(gt * log_p + (1.0 - gt) * log_1mp,
                   axis=(0, 1), keepdims=True)
    sq = (pred_adj - true_adj) ** 2
    bce_ref[0, :, :] = bce
    sq_ref[0, :, :] = jnp.sum(gt * sq, axis=(0, 1), keepdims=True)
    cnt_out_ref[0, :, :] = jnp.sum(gt, axis=(0, 1), keepdims=True)


@functools.partial(jax.jit)
def kernel(predictions, actions_label):
    idx_shape = jax.ShapeDtypeStruct((B, 1, T), jnp.int32)
    pidx, aidx = pl.pallas_call(
        _idx_body,
        grid=(B // NB,),
        in_specs=[
            pl.BlockSpec((NB, C, T), lambda b: (b, 0, 0)),
            pl.BlockSpec((NB, C, T), lambda b: (b, 0, 0)),
        ],
        out_specs=[
            pl.BlockSpec((NB, 1, T), lambda b: (b, 0, 0)),
            pl.BlockSpec((NB, 1, T), lambda b: (b, 0, 0)),
        ],
        out_shape=[idx_shape, idx_shape],
    )(predictions, actions_label)

    counts = _sc_hist_make()(pidx, aidx)
    counts4 = counts.reshape(B, 2, C, C)

    per_batch = jax.ShapeDtypeStruct((B, 1, 1), jnp.float32)
    bce_v, sq_v, cnt_v = pl.pallas_call(
        _loss_body,
        grid=(B,),
        in_specs=[pl.BlockSpec((1, 2, C, C), lambda b: (b, 0, 0, 0))],
        out_specs=[
            pl.BlockSpec((1, 1, 1), lambda b: (b, 0, 0)),
            pl.BlockSpec((1, 1, 1), lambda b: (b, 0, 0)),
            pl.BlockSpec((1, 1, 1), lambda b: (b, 0, 0)),
        ],
        out_shape=[per_batch, per_batch, per_batch],
    )(counts4)

    bce = jnp.sum(bce_v) / (B * C * C)
    cnt = jnp.sum(cnt_v)
    mse = jnp.sum(sq_v) / jnp.maximum(cnt, 1.0)
    return bce + jnp.where(cnt > 0, mse, 0.0)
